# Initial kernel scaffold; baseline (speedup 1.0000x reference)
import jax
import jax.numpy as jnp
from jax.experimental import pallas as pl


def kernel(z, objs, triples, attributes, params):
    raise NotImplementedError("write your pallas kernel here")


# speedup vs baseline: 2.6666x; 2.6666x over previous
import functools

import jax
import jax.numpy as jnp
from jax import lax
from jax.experimental import pallas as pl
from jax.experimental.pallas import tpu as pltpu
from jax.experimental.pallas import tpu_sc as plsc

EMB = 128
H = 512

NC = 2
NS = 16
NW = NC * NS
GCH = 80
SCH = 80
WCH = 80

@functools.cache
def _get_mesh():
    return plsc.VectorSubcoreMesh(
        core_axis_name="c", subcore_axis_name="s", num_cores=NC, num_subcores=NS)


def _sc_gather(table, idx):
    V, D = table.shape
    B = idx.shape[0]
    step = NW * GCH
    Bp = ((B + step - 1) // step) * step
    if Bp != B:
        idx = jnp.concatenate([idx, jnp.zeros((Bp - B,), jnp.int32)])
    per_w = Bp // NW
    nch = per_w // GCH

    @functools.partial(
        pl.kernel,
        out_type=jax.ShapeDtypeStruct((Bp, D), jnp.float32),
        mesh=_get_mesh(),
        scratch_types=[
            pltpu.VMEM((GCH,), jnp.int32),
            pltpu.VMEM((GCH, D), jnp.float32),
            pltpu.SemaphoreType.DMA,
        ],
    )
    def gk(table_h, idx_h, out_h, idx_v, rows_v, sem):
        wid = lax.axis_index("s") * NC + lax.axis_index("c")
        base = wid * per_w

        def body(i, carry):
            off = base + i * GCH
            pltpu.sync_copy(idx_h.at[pl.ds(off, GCH)], idx_v)
            pltpu.async_copy(table_h.at[idx_v], rows_v, sem).wait()
            pltpu.sync_copy(rows_v, out_h.at[pl.ds(off, GCH)])
            return carry

        lax.fori_loop(0, nch, body, 0)

    return gk(table, idx)


def _sc_scatter_pool(vals, s2d, o2d, n_nodes):
    E = vals.shape[1]
    ech = E // NS // SCH
    rpt = n_nodes // NS
    wbk = rpt // WCH
    zeros_src = jnp.zeros((SCH, EMB), jnp.float32)

    @functools.partial(
        pl.kernel,
        out_type=jax.ShapeDtypeStruct((n_nodes, H), jnp.float32),
        mesh=_get_mesh(),
        scratch_types=[
            pltpu.VMEM((ech, SCH), jnp.int32),
            pltpu.VMEM((ech, SCH), jnp.int32),
            pltpu.VMEM((SCH, EMB), jnp.float32),
            pltpu.VMEM_SHARED((n_nodes, EMB), jnp.float32),
            pltpu.SemaphoreType.DMA,
        ],
    )
    def sk(vals_h, s_h, o_h, z_h, out_h, sidx_v, oidx_v, vals_v, acc, sem):
        c = lax.axis_index("c")
        t = lax.axis_index("s")
        pltpu.sync_copy(s_h.at[t], sidx_v)
        pltpu.sync_copy(o_h.at[t], oidx_v)
        for cc in range(2):
            col0 = (c * 2 + cc) * EMB
            pltpu.sync_copy(z_h, vals_v)

            def zbody(k, carry):
                pltpu.sync_copy(vals_v, acc.at[pl.ds(t * rpt + k * WCH, WCH)])
                return carry

            lax.fori_loop(0, wbk, zbody, 0)
            plsc.subcore_barrier()

            def body(j, carry):
                row0 = t * (E // NS) + j * SCH
                pltpu.sync_copy(vals_h.at[0, pl.ds(row0, SCH), pl.ds(col0, EMB)], vals_v)
                pltpu.sync_copy(vals_v, acc.at[sidx_v.at[j]], add=True)
                pltpu.sync_copy(vals_h.at[1, pl.ds(row0, SCH), pl.ds(col0, EMB)], vals_v)
                pltpu.sync_copy(vals_v, acc.at[oidx_v.at[j]], add=True)
                return carry

            lax.fori_loop(0, ech, body, 0)
            plsc.subcore_barrier()

            def wb(k, carry):
                r0 = t * rpt + k * WCH
                pltpu.sync_copy(acc.at[pl.ds(r0, WCH)], vals_v)
                pltpu.sync_copy(vals_v, out_h.at[pl.ds(r0, WCH), pl.ds(col0, EMB)])
                return carry

            lax.fori_loop(0, wbk, wb, 0)
            plsc.subcore_barrier()

    return sk(vals, s2d, o2d, zeros_src)


def _sc_counts(s2d, o2d, n_nodes):
    ech = s2d.shape[1]
    rpt = n_nodes // NS
    wbk = rpt // WCH
    zeros_src = jnp.zeros((SCH, EMB), jnp.float32)
    ones_src = jnp.ones((SCH, EMB), jnp.float32)

    @functools.partial(
        pl.kernel,
        out_type=jax.ShapeDtypeStruct((n_nodes, EMB), jnp.float32),
        mesh=_get_mesh(),
        scratch_types=[
            pltpu.VMEM((ech, SCH), jnp.int32),
            pltpu.VMEM((ech, SCH), jnp.int32),
            pltpu.VMEM((SCH, EMB), jnp.float32),
            pltpu.VMEM_SHARED((n_nodes, EMB), jnp.float32),
            pltpu.SemaphoreType.DMA,
        ],
    )
    def ck(s_h, o_h, z_h, ones_h, out_h, sidx_v, oidx_v, ones_v, acc, sem):
        c = lax.axis_index("c")
        t = lax.axis_index("s")

        @pl.when(c == 0)
        def _():
            pltpu.sync_copy(s_h.at[t], sidx_v)
            pltpu.sync_copy(o_h.at[t], oidx_v)
            pltpu.sync_copy(z_h, ones_v)

            def zbody(k, carry):
                pltpu.sync_copy(ones_v, acc.at[pl.ds(t * rpt + k * WCH, WCH)])
                return carry

            lax.fori_loop(0, wbk, zbody, 0)
            plsc.subcore_barrier()
            pltpu.sync_copy(ones_h, ones_v)

            def body(j, carry):
                pltpu.sync_copy(ones_v, acc.at[sidx_v.at[j]], add=True)
                pltpu.sync_copy(ones_v, acc.at[oidx_v.at[j]], add=True)
                return carry

            lax.fori_loop(0, ech, body, 0)
            plsc.subcore_barrier()

            def wb(k, carry):
                r0 = t * rpt + k * WCH
                pltpu.sync_copy(acc.at[pl.ds(r0, WCH)], ones_v)
                pltpu.sync_copy(ones_v, out_h.at[pl.ds(r0, WCH)])
                return carry

            lax.fori_loop(0, wbk, wb, 0)

    return ck(s2d, o2d, zeros_src, ones_src)


def _pick_block(n, cands):
    for b in cands:
        if n % b == 0:
            return b
    raise ValueError(f"no block divides {n}")


def _edge_mlp(gso, pred, w1s, w1p, w1o, b1, w2, b2, E):
    BE = _pick_block(E, (2000, 1600, 800, 400))
    nb = E // BE

    def body(gs_r, go_r, pr_r, w1s_r, w1p_r, w1o_r, b1_r, w2_r, b2_r, vso_r, vp_r):
        x = (jnp.dot(gs_r[...], w1s_r[...], preferred_element_type=jnp.float32)
             + jnp.dot(pr_r[...], w1p_r[...], preferred_element_type=jnp.float32)
             + jnp.dot(go_r[...], w1o_r[...], preferred_element_type=jnp.float32)
             + b1_r[...])
        h = jnp.maximum(x, 0.0)
        tt = jnp.dot(h, w2_r[...], preferred_element_type=jnp.float32) + b2_r[...]
        tt = jnp.maximum(tt, 0.0)
        vso_r[0] = tt[:, :H]
        vp_r[...] = tt[:, H:H + EMB]
        vso_r[1] = tt[:, H + EMB:]

    return pl.pallas_call(
        body,
        grid=(nb,),
        in_specs=[
            pl.BlockSpec((BE, EMB), lambda i: (i, 0)),
            pl.BlockSpec((BE, EMB), lambda i, nb=nb: (nb + i, 0)),
            pl.BlockSpec((BE, EMB), lambda i: (i, 0)),
            pl.BlockSpec((EMB, H), lambda i: (0, 0)),
            pl.BlockSpec((EMB, H), lambda i: (0, 0)),
            pl.BlockSpec((EMB, H), lambda i: (0, 0)),
            pl.BlockSpec((1, H), lambda i: (0, 0)),
            pl.BlockSpec((H, 2 * H + EMB), lambda i: (0, 0)),
            pl.BlockSpec((1, 2 * H + EMB), lambda i: (0, 0)),
        ],
        out_specs=[
            pl.BlockSpec((2, BE, H), lambda i: (0, i, 0)),
            pl.BlockSpec((BE, EMB), lambda i: (i, 0)),
        ],
        out_shape=[
            jax.ShapeDtypeStruct((2, E, H), jnp.float32),
            jax.ShapeDtypeStruct((E, EMB), jnp.float32),
        ],
    )(gso, gso, pred, w1s, w1p, w1o, b1, w2, b2)


def _node_mlp(pooled, counts, v1, c1, v2, c2, N):
    BN = _pick_block(N, (2048, 2000, 1024, 1000, 512, 500))

    def body(p_r, cnt_r, v1_r, c1_r, v2_r, c2_r, o_r):
        rcp = 1.0 / jnp.maximum(cnt_r[:, 0:1], 1.0)
        x = p_r[...] * rcp
        h = jnp.maximum(jnp.dot(x, v1_r[...], preferred_element_type=jnp.float32) + c1_r[...], 0.0)
        o_r[...] = jnp.maximum(jnp.dot(h, v2_r[...], preferred_element_type=jnp.float32) + c2_r[...], 0.0)

    return pl.pallas_call(
        body,
        grid=(N // BN,),
        in_specs=[
            pl.BlockSpec((BN, H), lambda i: (i, 0)),
            pl.BlockSpec((BN, EMB), lambda i: (i, 0)),
            pl.BlockSpec((H, H), lambda i: (0, 0)),
            pl.BlockSpec((1, H), lambda i: (0, 0)),
            pl.BlockSpec((H, EMB), lambda i: (0, 0)),
            pl.BlockSpec((1, EMB), lambda i: (0, 0)),
        ],
        out_specs=pl.BlockSpec((BN, EMB), lambda i: (i, 0)),
        out_shape=jax.ShapeDtypeStruct((N, EMB), jnp.float32),
    )(pooled, counts, v1, c1, v2, c2)


def _heads(obj, z, ov0, wbo, wbz, wba, bb1, wb2, bb2, wao, waz, ba1, wa2, ba2, N):
    BN = _pick_block(N, (2048, 2000, 1024, 1000, 512, 500))

    def body(o_r, z_r, a_r, wbo_r, wbz_r, wba_r, bb1_r, wb2_r, bb2_r,
             wao_r, waz_r, ba1_r, wa2_r, ba2_r, box_r, ang_r):
        ob = o_r[...]
        zz = z_r[...]
        xb = (jnp.dot(ob, wbo_r[...], preferred_element_type=jnp.float32)
              + jnp.dot(zz, wbz_r[...], preferred_element_type=jnp.float32)
              + jnp.dot(a_r[...], wba_r[...], preferred_element_type=jnp.float32)
              + bb1_r[...])
        xb = jnp.maximum(xb, 0.0)
        box_r[...] = jnp.dot(xb, wb2_r[...], preferred_element_type=jnp.float32) + bb2_r[...]
        xa = (jnp.dot(ob, wao_r[...], preferred_element_type=jnp.float32)
              + jnp.dot(zz, waz_r[...], preferred_element_type=jnp.float32)
              + ba1_r[...])
        xa = jnp.maximum(xa, 0.0)
        lg = jnp.dot(xa, wa2_r[...], preferred_element_type=jnp.float32) + ba2_r[...]
        col = lax.broadcasted_iota(jnp.int32, (BN, EMB), 1)
        valid = col < 24
        m = jnp.max(jnp.where(valid, lg, -1e30), axis=1, keepdims=True)
        e = jnp.where(valid, jnp.exp(lg - m), 0.0)
        lse = jnp.log(jnp.sum(e, axis=1, keepdims=True))
        ang_r[...] = lg - m - lse

    wspec = pl.BlockSpec((EMB, H), lambda i: (0, 0))
    return pl.pallas_call(
        body,
        grid=(N // BN,),
        in_specs=[
            pl.BlockSpec((BN, EMB), lambda i: (i, 0)),
            pl.BlockSpec((BN, EMB), lambda i: (i, 0)),
            pl.BlockSpec((BN, EMB), lambda i: (i, 0)),
            wspec, wspec, wspec,
            pl.BlockSpec((1, H), lambda i: (0, 0)),
            pl.BlockSpec((H, EMB), lambda i: (0, 0)),
            pl.BlockSpec((1, EMB), lambda i: (0, 0)),
            wspec, wspec,
            pl.BlockSpec((1, H), lambda i: (0, 0)),
            pl.BlockSpec((H, EMB), lambda i: (0, 0)),
            pl.BlockSpec((1, EMB), lambda i: (0, 0)),
        ],
        out_specs=[
            pl.BlockSpec((BN, EMB), lambda i: (i, 0)),
            pl.BlockSpec((BN, EMB), lambda i: (i, 0)),
        ],
        out_shape=[
            jax.ShapeDtypeStruct((N, EMB), jnp.float32),
            jax.ShapeDtypeStruct((N, EMB), jnp.float32),
        ],
    )(obj, z, ov0, wbo, wbz, wba, bb1, wb2, bb2, wao, waz, ba1, wa2, ba2)


def kernel(z, objs, triples, attributes, params):
    N = z.shape[0]
    E = triples.shape[0]
    s = triples[:, 0]
    p = triples[:, 1]
    o = triples[:, 2]

    obj_emb = params['obj_emb']
    attr_emb = params['attr_emb']
    n_obj, d_obj = obj_emb.shape
    n_attr = attr_emb.shape[0]
    combo = jnp.concatenate([
        jnp.repeat(obj_emb, n_attr, axis=0),
        jnp.tile(attr_emb, (n_obj, 1)),
    ], axis=1)
    obj_vecs0 = _sc_gather(combo, objs * n_attr + attributes)
    pred = _sc_gather(params['pred_emb'], p)

    so = jnp.concatenate([s, o])
    s2d = s.reshape(NS, E // NS // SCH, SCH)
    o2d = o.reshape(NS, E // NS // SCH, SCH)
    NP = -(-N // (NS * WCH)) * (NS * WCH)
    counts = _sc_counts(s2d, o2d, NP)

    obj_vecs = obj_vecs0
    for layer in params['gconv']:
        (w1, b1), (w2, b2) = layer['net1']
        (v1, c1), (v2, c2) = layer['net2']
        gso = _sc_gather(obj_vecs, so)
        vals, pred = _edge_mlp(
            gso, pred,
            w1[:EMB], w1[EMB:2 * EMB], w1[2 * EMB:],
            b1.reshape(1, H), w2, b2.reshape(1, 2 * H + EMB), E)
        pooled = _sc_scatter_pool(vals, s2d, o2d, NP)
        obj_vecs = _node_mlp(pooled, counts, v1, c1.reshape(1, H),
                             v2, c2.reshape(1, EMB), NP)

    (wb1, bb1), (wb2, bb2) = params['box_net']
    (wa1, ba1), (wa2, ba2) = params['angle_net']
    wba = jnp.concatenate([jnp.zeros((96, H), jnp.float32), wb1[2 * EMB:]], axis=0)
    wb2p = jnp.pad(wb2, ((0, 0), (0, EMB - 6)))
    bb2p = jnp.pad(bb2, (0, EMB - 6)).reshape(1, EMB)
    wa2p = jnp.pad(wa2, ((0, 0), (0, EMB - 24)))
    ba2p = jnp.pad(ba2, (0, EMB - 24)).reshape(1, EMB)
    boxes, angles = _heads(
        obj_vecs, z, obj_vecs0,
        wb1[:EMB], wb1[EMB:2 * EMB], wba, bb1.reshape(1, H), wb2p, bb2p,
        wa1[:EMB], wa1[EMB:2 * EMB], ba1.reshape(1, H), wa2p, ba2p, N)
    return boxes[:, :6], angles[:, :24]


# speedup vs baseline: 4.2581x; 1.5968x over previous
import functools

import jax
import jax.numpy as jnp
from jax import lax
from jax.experimental import pallas as pl
from jax.experimental.pallas import tpu as pltpu
from jax.experimental.pallas import tpu_sc as plsc

EMB = 128
H = 512

NC = 2
NS = 16
NW = NC * NS
GCH = 80
SCH = 80
WCH = 80

@functools.cache
def _get_mesh():
    return plsc.VectorSubcoreMesh(
        core_axis_name="c", subcore_axis_name="s", num_cores=NC, num_subcores=NS)


def _ring_params(nch):
    if nch % 5 == 0 and nch >= 10:
        return 5, 2
    if nch % 4 == 0:
        return 4, 2
    if nch % 2 == 0:
        return 2, 1
    return 1, 0


def _sc_gather(table, idx):
    V, D = table.shape
    B = idx.shape[0]
    step = NW * GCH
    Bp = ((B + step - 1) // step) * step
    if Bp != B:
        idx = jnp.concatenate([idx, jnp.zeros((Bp - B,), jnp.int32)])
    per_w = Bp // NW
    nch = per_w // GCH
    NB, L = _ring_params(nch)

    @functools.partial(
        pl.kernel,
        out_type=jax.ShapeDtypeStruct((Bp, D), jnp.float32),
        mesh=_get_mesh(),
        scratch_types=[
            pltpu.VMEM((per_w,), jnp.int32),
            pltpu.VMEM((NB * GCH, D), jnp.float32),
            pltpu.SemaphoreType.DMA,
            pltpu.SemaphoreType.DMA,
        ],
    )
    def gk(table_h, idx_h, out_h, idxall_v, rows_v, gsem, wsem):
        wid = lax.axis_index("s") * NC + lax.axis_index("c")
        base = wid * per_w
        pltpu.sync_copy(idx_h.at[pl.ds(base, per_w)], idxall_v)

        def start_gather(g, buf):
            pltpu.async_copy(
                table_h.at[idxall_v.at[pl.ds(g * GCH, GCH)]],
                rows_v.at[pl.ds(buf * GCH, GCH)], gsem)

        def drain_write():
            pltpu.make_async_copy(
                rows_v.at[pl.ds(0, GCH)],
                out_h.at[pl.ds(base, GCH)], wsem).wait()

        for b in range(L):
            start_gather(b, b)

        def outer(i0, carry):
            for b in range(NB):
                i = i0 * NB + b

                @pl.when(i >= NB - L)
                def _():
                    drain_write()

                g = i + L
                bg = (b + L) % NB

                @pl.when(g < nch)
                def _():
                    start_gather(g, bg)

                pltpu.make_async_copy(
                    table_h.at[idxall_v.at[pl.ds(0, GCH)]],
                    rows_v.at[pl.ds(b * GCH, GCH)], gsem).wait()
                pltpu.async_copy(
                    rows_v.at[pl.ds(b * GCH, GCH)],
                    out_h.at[pl.ds(base + i * GCH, GCH)], wsem)
            return carry

        lax.fori_loop(0, nch // NB, outer, 0)
        for _ in range(min(nch, NB - L)):
            drain_write()

    return gk(table, idx)


def _sc_scatter_pool(vals, s2d, o2d, n_nodes):
    E = vals.shape[1]
    ech = E // NS // SCH
    rpt = n_nodes // NS
    wbk = rpt // WCH
    NB, L = 4, 2
    main = (ech // NB) * NB
    zeros_src = jnp.zeros((WCH, EMB), jnp.float32)

    @functools.partial(
        pl.kernel,
        out_type=jax.ShapeDtypeStruct((n_nodes, H), jnp.float32),
        mesh=_get_mesh(),
        scratch_types=[
            pltpu.VMEM((NB, 1, SCH), jnp.int32),
            pltpu.VMEM((NB * SCH, EMB), jnp.float32),
            pltpu.VMEM_SHARED((n_nodes, EMB), jnp.float32),
            pltpu.SemaphoreType.DMA,
            pltpu.SemaphoreType.DMA,
            pltpu.SemaphoreType.DMA,
        ],
    )
    def sk(vals_h, s_h, o_h, z_h, out_h, idxr_v, vals_v, acc,
           isem, lsem, ssem):
        c = lax.axis_index("c")
        t = lax.axis_index("s")

        def drain_scatter():
            pltpu.make_async_copy(
                vals_v.at[pl.ds(0, SCH)],
                acc.at[pl.ds(0, SCH)], ssem).wait()

        for cc in range(2):
            col0 = (c * 2 + cc) * EMB
            pltpu.sync_copy(z_h, vals_v.at[pl.ds(0, WCH)])

            def zbody(k, carry):
                pltpu.sync_copy(vals_v.at[pl.ds(0, WCH)],
                                acc.at[pl.ds(t * rpt + k * WCH, WCH)])
                return carry

            lax.fori_loop(0, wbk, zbody, 0)
            plsc.subcore_barrier()

            for src_i, idx_h in ((0, s_h), (1, o_h)):
                def start_step(g, buf, src_i=src_i, idx_h=idx_h):
                    pltpu.async_copy(idx_h.at[g, t], idxr_v.at[buf], isem)
                    pltpu.async_copy(
                        vals_h.at[src_i, pl.ds(t * (E // NS) + g * SCH, SCH),
                                  pl.ds(col0, EMB)],
                        vals_v.at[pl.ds(buf * SCH, SCH)], lsem)

                for b in range(min(L, main)):
                    start_step(b, b)

                def outer(i0, carry, idx_h=idx_h, start_step=start_step):
                    for b in range(NB):
                        j = i0 * NB + b

                        @pl.when(j >= NB - L)
                        def _():
                            drain_scatter()

                        g = j + L
                        bg = (b + L) % NB

                        @pl.when(g < main)
                        def _():
                            start_step(g, bg)

                        pltpu.make_async_copy(
                            idx_h.at[0, t], idxr_v.at[b], isem).wait()
                        pltpu.make_async_copy(
                            vals_h.at[0, pl.ds(0, SCH), pl.ds(0, EMB)],
                            vals_v.at[pl.ds(b * SCH, SCH)], lsem).wait()
                        pltpu.async_copy(
                            vals_v.at[pl.ds(b * SCH, SCH)],
                            acc.at[idxr_v.at[b, 0]], ssem, add=True)
                    return carry

                lax.fori_loop(0, main // NB, outer, 0)
                for _ in range(min(main, NB - L)):
                    drain_scatter()
                for jt in range(main, ech):
                    pltpu.sync_copy(idx_h.at[jt, t], idxr_v.at[0])
                    pltpu.sync_copy(
                        vals_h.at[src_i, pl.ds(t * (E // NS) + jt * SCH, SCH),
                                  pl.ds(col0, EMB)],
                        vals_v.at[pl.ds(0, SCH)])
                    pltpu.sync_copy(vals_v.at[pl.ds(0, SCH)],
                                    acc.at[idxr_v.at[0, 0]], add=True)

            plsc.subcore_barrier()

            def wb(k, carry):
                r0 = t * rpt + k * WCH
                pltpu.sync_copy(acc.at[pl.ds(r0, WCH)], vals_v.at[pl.ds(0, WCH)])
                pltpu.sync_copy(vals_v.at[pl.ds(0, WCH)],
                                out_h.at[pl.ds(r0, WCH), pl.ds(col0, EMB)])
                return carry

            lax.fori_loop(0, wbk, wb, 0)
            plsc.subcore_barrier()

    return sk(vals, s2d, o2d, zeros_src)


def _sc_counts(s2d, o2d, n_nodes):
    ech = s2d.shape[1]
    rpt = n_nodes // NS
    wbk = rpt // WCH
    zeros_src = jnp.zeros((SCH, EMB), jnp.float32)
    ones_src = jnp.ones((SCH, EMB), jnp.float32)

    @functools.partial(
        pl.kernel,
        out_type=jax.ShapeDtypeStruct((n_nodes, EMB), jnp.float32),
        mesh=_get_mesh(),
        scratch_types=[
            pltpu.VMEM((ech, SCH), jnp.int32),
            pltpu.VMEM((ech, SCH), jnp.int32),
            pltpu.VMEM((SCH, EMB), jnp.float32),
            pltpu.VMEM_SHARED((n_nodes, EMB), jnp.float32),
            pltpu.SemaphoreType.DMA,
        ],
    )
    def ck(s_h, o_h, z_h, ones_h, out_h, sidx_v, oidx_v, ones_v, acc, csem):
        c = lax.axis_index("c")
        t = lax.axis_index("s")

        @pl.when(c == 0)
        def _():
            pltpu.sync_copy(s_h.at[t], sidx_v)
            pltpu.sync_copy(o_h.at[t], oidx_v)
            pltpu.sync_copy(z_h, ones_v)

            def zbody(k, carry):
                pltpu.sync_copy(ones_v, acc.at[pl.ds(t * rpt + k * WCH, WCH)])
                return carry

            lax.fori_loop(0, wbk, zbody, 0)
            plsc.subcore_barrier()
            pltpu.sync_copy(ones_h, ones_v)

            def body(j, carry):
                pltpu.async_copy(ones_v, acc.at[sidx_v.at[j]], csem, add=True)
                pltpu.async_copy(ones_v, acc.at[oidx_v.at[j]], csem, add=True)
                return carry

            lax.fori_loop(0, ech, body, 0)

            def drain(j, carry):
                pltpu.make_async_copy(
                    ones_v, acc.at[pl.ds(0, SCH)], csem).wait()
                pltpu.make_async_copy(
                    ones_v, acc.at[pl.ds(0, SCH)], csem).wait()
                return carry

            lax.fori_loop(0, ech, drain, 0)
            plsc.subcore_barrier()

            def wb(k, carry):
                r0 = t * rpt + k * WCH
                pltpu.sync_copy(acc.at[pl.ds(r0, WCH)], ones_v)
                pltpu.sync_copy(ones_v, out_h.at[pl.ds(r0, WCH)])
                return carry

            lax.fori_loop(0, wbk, wb, 0)

    return ck(s2d, o2d, zeros_src, ones_src)


def _pick_block(n, cands):
    for b in cands:
        if n % b == 0:
            return b
    raise ValueError(f"no block divides {n}")


def _edge_mlp(gso, pred, w1s, w1p, w1o, b1, w2, b2, E):
    BE = _pick_block(E, (2000, 1600, 800, 400))
    nb = E // BE

    def body(gs_r, go_r, pr_r, w1s_r, w1p_r, w1o_r, b1_r, w2_r, b2_r, vso_r, vp_r):
        x = (jnp.dot(gs_r[...], w1s_r[...], preferred_element_type=jnp.float32)
             + jnp.dot(pr_r[...], w1p_r[...], preferred_element_type=jnp.float32)
             + jnp.dot(go_r[...], w1o_r[...], preferred_element_type=jnp.float32)
             + b1_r[...])
        h = jnp.maximum(x, 0.0)
        tt = jnp.dot(h, w2_r[...], preferred_element_type=jnp.float32) + b2_r[...]
        tt = jnp.maximum(tt, 0.0)
        vso_r[0] = tt[:, :H]
        vp_r[...] = tt[:, H:H + EMB]
        vso_r[1] = tt[:, H + EMB:]

    return pl.pallas_call(
        body,
        grid=(nb,),
        in_specs=[
            pl.BlockSpec((BE, EMB), lambda i: (i, 0)),
            pl.BlockSpec((BE, EMB), lambda i, nb=nb: (nb + i, 0)),
            pl.BlockSpec((BE, EMB), lambda i: (i, 0)),
            pl.BlockSpec((EMB, H), lambda i: (0, 0)),
            pl.BlockSpec((EMB, H), lambda i: (0, 0)),
            pl.BlockSpec((EMB, H), lambda i: (0, 0)),
            pl.BlockSpec((1, H), lambda i: (0, 0)),
            pl.BlockSpec((H, 2 * H + EMB), lambda i: (0, 0)),
            pl.BlockSpec((1, 2 * H + EMB), lambda i: (0, 0)),
        ],
        out_specs=[
            pl.BlockSpec((2, BE, H), lambda i: (0, i, 0)),
            pl.BlockSpec((BE, EMB), lambda i: (i, 0)),
        ],
        out_shape=[
            jax.ShapeDtypeStruct((2, E, H), jnp.float32),
            jax.ShapeDtypeStruct((E, EMB), jnp.float32),
        ],
    )(gso, gso, pred, w1s, w1p, w1o, b1, w2, b2)


def _node_mlp(pooled, counts, v1, c1, v2, c2, N):
    BN = _pick_block(N, (2048, 2000, 1024, 1000, 512, 500))

    def body(p_r, cnt_r, v1_r, c1_r, v2_r, c2_r, o_r):
        rcp = 1.0 / jnp.maximum(cnt_r[:, 0:1], 1.0)
        x = p_r[...] * rcp
        h = jnp.maximum(jnp.dot(x, v1_r[...], preferred_element_type=jnp.float32) + c1_r[...], 0.0)
        o_r[...] = jnp.maximum(jnp.dot(h, v2_r[...], preferred_element_type=jnp.float32) + c2_r[...], 0.0)

    return pl.pallas_call(
        body,
        grid=(N // BN,),
        in_specs=[
            pl.BlockSpec((BN, H), lambda i: (i, 0)),
            pl.BlockSpec((BN, EMB), lambda i: (i, 0)),
            pl.BlockSpec((H, H), lambda i: (0, 0)),
            pl.BlockSpec((1, H), lambda i: (0, 0)),
            pl.BlockSpec((H, EMB), lambda i: (0, 0)),
            pl.BlockSpec((1, EMB), lambda i: (0, 0)),
        ],
        out_specs=pl.BlockSpec((BN, EMB), lambda i: (i, 0)),
        out_shape=jax.ShapeDtypeStruct((N, EMB), jnp.float32),
    )(pooled, counts, v1, c1, v2, c2)


def _heads(obj, z, ov0, wbo, wbz, wba, bb1, wb2, bb2, wao, waz, ba1, wa2, ba2, N):
    BN = _pick_block(N, (2048, 2000, 1024, 1000, 512, 500))

    def body(o_r, z_r, a_r, wbo_r, wbz_r, wba_r, bb1_r, wb2_r, bb2_r,
             wao_r, waz_r, ba1_r, wa2_r, ba2_r, box_r, ang_r):
        ob = o_r[...]
        zz = z_r[...]
        xb = (jnp.dot(ob, wbo_r[...], preferred_element_type=jnp.float32)
              + jnp.dot(zz, wbz_r[...], preferred_element_type=jnp.float32)
              + jnp.dot(a_r[...], wba_r[...], preferred_element_type=jnp.float32)
              + bb1_r[...])
        xb = jnp.maximum(xb, 0.0)
        box_r[...] = jnp.dot(xb, wb2_r[...], preferred_element_type=jnp.float32) + bb2_r[...]
        xa = (jnp.dot(ob, wao_r[...], preferred_element_type=jnp.float32)
              + jnp.dot(zz, waz_r[...], preferred_element_type=jnp.float32)
              + ba1_r[...])
        xa = jnp.maximum(xa, 0.0)
        lg = jnp.dot(xa, wa2_r[...], preferred_element_type=jnp.float32) + ba2_r[...]
        col = lax.broadcasted_iota(jnp.int32, (BN, EMB), 1)
        valid = col < 24
        m = jnp.max(jnp.where(valid, lg, -1e30), axis=1, keepdims=True)
        e = jnp.where(valid, jnp.exp(lg - m), 0.0)
        lse = jnp.log(jnp.sum(e, axis=1, keepdims=True))
        ang_r[...] = lg - m - lse

    wspec = pl.BlockSpec((EMB, H), lambda i: (0, 0))
    return pl.pallas_call(
        body,
        grid=(N // BN,),
        in_specs=[
            pl.BlockSpec((BN, EMB), lambda i: (i, 0)),
            pl.BlockSpec((BN, EMB), lambda i: (i, 0)),
            pl.BlockSpec((BN, EMB), lambda i: (i, 0)),
            wspec, wspec, wspec,
            pl.BlockSpec((1, H), lambda i: (0, 0)),
            pl.BlockSpec((H, EMB), lambda i: (0, 0)),
            pl.BlockSpec((1, EMB), lambda i: (0, 0)),
            wspec, wspec,
            pl.BlockSpec((1, H), lambda i: (0, 0)),
            pl.BlockSpec((H, EMB), lambda i: (0, 0)),
            pl.BlockSpec((1, EMB), lambda i: (0, 0)),
        ],
        out_specs=[
            pl.BlockSpec((BN, EMB), lambda i: (i, 0)),
            pl.BlockSpec((BN, EMB), lambda i: (i, 0)),
        ],
        out_shape=[
            jax.ShapeDtypeStruct((N, EMB), jnp.float32),
            jax.ShapeDtypeStruct((N, EMB), jnp.float32),
        ],
    )(obj, z, ov0, wbo, wbz, wba, bb1, wb2, bb2, wao, waz, ba1, wa2, ba2)


def kernel(z, objs, triples, attributes, params):
    N = z.shape[0]
    E = triples.shape[0]
    s = triples[:, 0]
    p = triples[:, 1]
    o = triples[:, 2]

    obj_emb = params['obj_emb']
    attr_emb = params['attr_emb']
    n_obj, d_obj = obj_emb.shape
    n_attr = attr_emb.shape[0]
    combo = jnp.concatenate([
        jnp.repeat(obj_emb, n_attr, axis=0),
        jnp.tile(attr_emb, (n_obj, 1)),
    ], axis=1)
    obj_vecs0 = _sc_gather(combo, objs * n_attr + attributes)
    pred = _sc_gather(params['pred_emb'], p)

    so = jnp.concatenate([s, o])
    ech = E // NS // SCH
    s2d = s.reshape(NS, ech, SCH)
    o2d = o.reshape(NS, ech, SCH)
    s4d = s2d.transpose(1, 0, 2).reshape(ech, NS, 1, SCH)
    o4d = o2d.transpose(1, 0, 2).reshape(ech, NS, 1, SCH)
    NP = -(-N // (NS * WCH)) * (NS * WCH)
    counts = _sc_counts(s2d, o2d, NP)

    obj_vecs = obj_vecs0
    for layer in params['gconv']:
        (w1, b1), (w2, b2) = layer['net1']
        (v1, c1), (v2, c2) = layer['net2']
        gso = _sc_gather(obj_vecs, so)
        vals, pred = _edge_mlp(
            gso, pred,
            w1[:EMB], w1[EMB:2 * EMB], w1[2 * EMB:],
            b1.reshape(1, H), w2, b2.reshape(1, 2 * H + EMB), E)
        pooled = _sc_scatter_pool(vals, s4d, o4d, NP)
        obj_vecs = _node_mlp(pooled, counts, v1, c1.reshape(1, H),
                             v2, c2.reshape(1, EMB), NP)

    (wb1, bb1), (wb2, bb2) = params['box_net']
    (wa1, ba1), (wa2, ba2) = params['angle_net']
    wba = jnp.concatenate([jnp.zeros((96, H), jnp.float32), wb1[2 * EMB:]], axis=0)
    wb2p = jnp.pad(wb2, ((0, 0), (0, EMB - 6)))
    bb2p = jnp.pad(bb2, (0, EMB - 6)).reshape(1, EMB)
    wa2p = jnp.pad(wa2, ((0, 0), (0, EMB - 24)))
    ba2p = jnp.pad(ba2, (0, EMB - 24)).reshape(1, EMB)
    boxes, angles = _heads(
        obj_vecs, z, obj_vecs0,
        wb1[:EMB], wb1[EMB:2 * EMB], wba, bb1.reshape(1, H), wb2p, bb2p,
        wa1[:EMB], wa1[EMB:2 * EMB], ba1.reshape(1, H), wa2p, ba2p, N)
    return boxes[:, :6], angles[:, :24]
